# baseline (device time: 351588 ns/iter reference)
import jax
import jax.numpy as jnp
from jax import lax
from jax.experimental import pallas as pl
from jax.experimental.pallas import tpu as pltpu

N_DEV = 4
BLK = 64


def kernel(x, Wq, K_ext, V_ext, Wo):
    B, Sq, Dm = x.shape
    Skv_loc = K_ext.shape[1]
    Dh = K_ext.shape[3]
    Hl = Wq.shape[1] // Dh
    scale = 0.125

    def body(x_ref, wq_ref, k_ref, v_ref, wo_ref, out_ref, obuf,
             kbuf, vbuf, ostage, lsum, acc,
             copy_sems, ksend, krecv, vsend, vrecv, osend, orecv):
        my = lax.axis_index("i")

        kcp = pltpu.make_async_copy(
            k_ref.at[:, :, pl.ds(my * Hl, Hl), :], kbuf.at[N_DEV - 1],
            copy_sems.at[0])
        vcp = pltpu.make_async_copy(
            v_ref.at[:, :, pl.ds(my * Hl, Hl), :], vbuf.at[N_DEV - 1],
            copy_sems.at[1])
        kcp.start()
        vcp.start()

        kv_rdmas = []
        for off in range(1, N_DEV):
            d = lax.rem(my + off, N_DEV)
            kr = pltpu.make_async_remote_copy(
                src_ref=k_ref.at[:, :, pl.ds(d * Hl, Hl), :],
                dst_ref=kbuf.at[off - 1],
                send_sem=ksend.at[off - 1], recv_sem=krecv.at[off - 1],
                device_id=(d,), device_id_type=pl.DeviceIdType.MESH)
            vr = pltpu.make_async_remote_copy(
                src_ref=v_ref.at[:, :, pl.ds(d * Hl, Hl), :],
                dst_ref=vbuf.at[off - 1],
                send_sem=vsend.at[off - 1], recv_sem=vrecv.at[off - 1],
                device_id=(d,), device_id_type=pl.DeviceIdType.MESH)
            kr.start()
            vr.start()
            kv_rdmas.append((kr, vr))

        Qs = [jnp.dot(x_ref[b], wq_ref[...], preferred_element_type=jnp.float32)
              for b in range(B)]

        def fold_slot(slot, first):
            src = lax.rem(my + N_DEV - 1 - slot, N_DEV)
            qb = lax.broadcasted_iota(jnp.int32, (Sq, Skv_loc), 0) // BLK
            kj = lax.broadcasted_iota(jnp.int32, (Sq, Skv_loc), 1) + src * Skv_loc
            kb = kj // BLK
            mask = (qb == kb) | (kb == 0) | (lax.rem(qb + kb, 3) == 0)
            for b in range(B):
                for h in range(Hl):
                    q_h = Qs[b][:, h * Dh:(h + 1) * Dh]
                    sp = lax.dot_general(
                        q_h, kbuf[slot, b, :, h, :],
                        (((1,), (1,)), ((), ())),
                        preferred_element_type=jnp.float32) * scale
                    e = jnp.where(mask, jnp.exp(sp), 0.0)
                    pv = jnp.dot(e, vbuf[slot, b, :, h, :],
                                 preferred_element_type=jnp.float32)
                    if first:
                        lsum[b, h] = jnp.sum(e, axis=1)
                        acc[b, h] = pv
                    else:
                        lsum[b, h] += jnp.sum(e, axis=1)
                        acc[b, h] += pv

        kcp.wait()
        vcp.wait()
        fold_slot(N_DEV - 1, first=True)
        for slot in (0, 2, 1):
            kv_rdmas[slot][0].wait_recv()
            kv_rdmas[slot][1].wait_recv()
            fold_slot(slot, first=False)

        o_rdmas = []
        for b in range(B):
            ctx = jnp.concatenate(
                [acc[b, h] / lsum[b, h][:, None] for h in range(Hl)],
                axis=1)
            out_ref[b] = jnp.dot(
                ctx, wo_ref[...], preferred_element_type=jnp.float32)
            for off in range(1, N_DEV):
                d = lax.rem(my + off, N_DEV)
                orr = pltpu.make_async_remote_copy(
                    src_ref=out_ref.at[b],
                    dst_ref=obuf.at[off - 1, b],
                    send_sem=osend.at[b, off - 1],
                    recv_sem=orecv.at[b, off - 1],
                    device_id=(d,), device_id_type=pl.DeviceIdType.MESH)
                orr.start()
                o_rdmas.append(orr)

        for kr, vr in kv_rdmas:
            kr.wait_send()
            vr.wait_send()

        for orr in o_rdmas:
            orr.wait()

        for j in range(N_DEV - 1):
            cp = pltpu.make_async_copy(obuf.at[j], ostage, copy_sems.at[0])
            cp.start()
            cp.wait()
            out_ref[...] = out_ref[...] + ostage[...]

    out, _ = pl.pallas_call(
        body,
        out_shape=[
            jax.ShapeDtypeStruct((B, Sq, Dm), jnp.float32),
            jax.ShapeDtypeStruct((N_DEV - 1, B, Sq, Dm), jnp.float32),
        ],
        in_specs=[
            pl.BlockSpec(memory_space=pltpu.VMEM),
            pl.BlockSpec(memory_space=pltpu.VMEM),
            pl.BlockSpec(memory_space=pltpu.HBM),
            pl.BlockSpec(memory_space=pltpu.HBM),
            pl.BlockSpec(memory_space=pltpu.VMEM),
        ],
        out_specs=[
            pl.BlockSpec(memory_space=pltpu.VMEM),
            pl.BlockSpec(memory_space=pltpu.HBM),
        ],
        scratch_shapes=[
            pltpu.VMEM((N_DEV, B, Skv_loc, Hl, Dh), jnp.float32),
            pltpu.VMEM((N_DEV, B, Skv_loc, Hl, Dh), jnp.float32),
            pltpu.VMEM((B, Sq, Dm), jnp.float32),
            pltpu.VMEM((B, Hl, Sq), jnp.float32),
            pltpu.VMEM((B, Hl, Sq, Dh), jnp.float32),
            pltpu.SemaphoreType.DMA((2,)),
            pltpu.SemaphoreType.DMA((N_DEV - 1,)),
            pltpu.SemaphoreType.DMA((N_DEV - 1,)),
            pltpu.SemaphoreType.DMA((N_DEV - 1,)),
            pltpu.SemaphoreType.DMA((N_DEV - 1,)),
            pltpu.SemaphoreType.DMA((B, N_DEV - 1)),
            pltpu.SemaphoreType.DMA((B, N_DEV - 1)),
        ],
        compiler_params=pltpu.CompilerParams(
            vmem_limit_bytes=64 * 1024 * 1024,
        ),
    )(x, Wq, K_ext, V_ext, Wo)
    return out


# device time: 229004 ns/iter; 1.5353x vs baseline; 1.5353x over previous
import os

import jax
import jax.numpy as jnp
from jax import lax
from jax.experimental import pallas as pl
from jax.experimental.pallas import tpu as pltpu

N_DEV = 4
BLK = 64
KMODE = os.environ.get("KMODE", "full")


def kernel(x, Wq, K_ext, V_ext, Wo):
    B, Sq, Dm = x.shape
    Skv_loc = K_ext.shape[1]
    Dh = K_ext.shape[3]
    Hl = Wq.shape[1] // Dh
    HD = Hl * Dh
    scale = 0.125

    K2 = K_ext.reshape(B, Skv_loc, K_ext.shape[2] * Dh)
    V2 = V_ext.reshape(B, Skv_loc, V_ext.shape[2] * Dh)

    def body(x_ref, wq_ref, k_ref, v_ref, wo_ref, out_ref, obuf,
             kbuf, vbuf, ostage, lsum, acc,
             copy_sems, ksend, krecv, vsend, vrecv, osend, orecv):
        my = lax.axis_index("i")

        kcp = pltpu.make_async_copy(
            k_ref.at[:, :, pl.ds(my * HD, HD)], kbuf.at[N_DEV - 1],
            copy_sems.at[0])
        vcp = pltpu.make_async_copy(
            v_ref.at[:, :, pl.ds(my * HD, HD)], vbuf.at[N_DEV - 1],
            copy_sems.at[1])
        kcp.start()
        vcp.start()

        kv_rdmas = []
        for off in range(1, N_DEV):
            d = lax.rem(my + off, N_DEV)
            kr = pltpu.make_async_remote_copy(
                src_ref=k_ref.at[:, :, pl.ds(d * HD, HD)],
                dst_ref=kbuf.at[off - 1],
                send_sem=ksend.at[off - 1], recv_sem=krecv.at[off - 1],
                device_id=(d,), device_id_type=pl.DeviceIdType.MESH)
            vr = pltpu.make_async_remote_copy(
                src_ref=v_ref.at[:, :, pl.ds(d * HD, HD)],
                dst_ref=vbuf.at[off - 1],
                send_sem=vsend.at[off - 1], recv_sem=vrecv.at[off - 1],
                device_id=(d,), device_id_type=pl.DeviceIdType.MESH)
            kr.start()
            vr.start()
            kv_rdmas.append((kr, vr))

        Qs = [jnp.dot(x_ref[b], wq_ref[...], preferred_element_type=jnp.float32)
              for b in range(B)]

        def fold_slot(slot, first):
            src = lax.rem(my + N_DEV - 1 - slot, N_DEV)
            qb = lax.broadcasted_iota(jnp.int32, (Sq, Skv_loc), 0) // BLK
            kj = lax.broadcasted_iota(jnp.int32, (Sq, Skv_loc), 1) + src * Skv_loc
            kb = kj // BLK
            mask = (qb == kb) | (kb == 0) | (lax.rem(qb + kb, 3) == 0)
            for b in range(B):
                for h in range(Hl):
                    q_h = Qs[b][:, h * Dh:(h + 1) * Dh]
                    sp = lax.dot_general(
                        q_h, kbuf[slot, b, :, h * Dh:(h + 1) * Dh],
                        (((1,), (1,)), ((), ())),
                        preferred_element_type=jnp.float32) * scale
                    e = jnp.where(mask, jnp.exp(sp), 0.0)
                    pv = jnp.dot(e, vbuf[slot, b, :, h * Dh:(h + 1) * Dh],
                                 preferred_element_type=jnp.float32)
                    if first:
                        lsum[b, h] = jnp.sum(e, axis=1)
                        acc[b, h] = pv
                    else:
                        lsum[b, h] += jnp.sum(e, axis=1)
                        acc[b, h] += pv

        kcp.wait()
        vcp.wait()
        fold_slot(N_DEV - 1, first=True)
        for slot in (0, 2, 1):
            kv_rdmas[slot][0].wait_recv()
            kv_rdmas[slot][1].wait_recv()
            if KMODE != "nocompute":
                fold_slot(slot, first=False)

        send_out = KMODE != "noout"
        o_rdmas = []
        for b in range(B):
            ctx = jnp.concatenate(
                [acc[b, h] / lsum[b, h][:, None] for h in range(Hl)],
                axis=1)
            out_ref[b] = jnp.dot(
                ctx, wo_ref[...], preferred_element_type=jnp.float32)
            for off in range(1, N_DEV) if send_out else []:
                d = lax.rem(my + off, N_DEV)
                orr = pltpu.make_async_remote_copy(
                    src_ref=out_ref.at[b],
                    dst_ref=obuf.at[off - 1, b],
                    send_sem=osend.at[b, off - 1],
                    recv_sem=orecv.at[b, off - 1],
                    device_id=(d,), device_id_type=pl.DeviceIdType.MESH)
                orr.start()
                o_rdmas.append(orr)

        for kr, vr in kv_rdmas:
            kr.wait_send()
            vr.wait_send()

        for orr in o_rdmas:
            orr.wait()

        for j in range(N_DEV - 1) if send_out else []:
            cp = pltpu.make_async_copy(obuf.at[j], ostage, copy_sems.at[0])
            cp.start()
            cp.wait()
            out_ref[...] = out_ref[...] + ostage[...]

    out, _ = pl.pallas_call(
        body,
        out_shape=[
            jax.ShapeDtypeStruct((B, Sq, Dm), jnp.float32),
            jax.ShapeDtypeStruct((N_DEV - 1, B, Sq, Dm), jnp.float32),
        ],
        in_specs=[
            pl.BlockSpec(memory_space=pltpu.VMEM),
            pl.BlockSpec(memory_space=pltpu.VMEM),
            pl.BlockSpec(memory_space=pltpu.VMEM),
            pl.BlockSpec(memory_space=pltpu.VMEM),
            pl.BlockSpec(memory_space=pltpu.VMEM),
        ],
        out_specs=[
            pl.BlockSpec(memory_space=pltpu.VMEM),
            pl.BlockSpec(memory_space=pltpu.HBM),
        ],
        scratch_shapes=[
            pltpu.VMEM((N_DEV, B, Skv_loc, HD), jnp.float32),
            pltpu.VMEM((N_DEV, B, Skv_loc, HD), jnp.float32),
            pltpu.VMEM((B, Sq, Dm), jnp.float32),
            pltpu.VMEM((B, Hl, Sq), jnp.float32),
            pltpu.VMEM((B, Hl, Sq, Dh), jnp.float32),
            pltpu.SemaphoreType.DMA((2,)),
            pltpu.SemaphoreType.DMA((N_DEV - 1,)),
            pltpu.SemaphoreType.DMA((N_DEV - 1,)),
            pltpu.SemaphoreType.DMA((N_DEV - 1,)),
            pltpu.SemaphoreType.DMA((N_DEV - 1,)),
            pltpu.SemaphoreType.DMA((B, N_DEV - 1)),
            pltpu.SemaphoreType.DMA((B, N_DEV - 1)),
        ],
        compiler_params=pltpu.CompilerParams(
            vmem_limit_bytes=64 * 1024 * 1024,
        ),
    )(x, Wq, K2, V2, Wo)
    return out


# device time: 137672 ns/iter; 2.5538x vs baseline; 1.6634x over previous
import os

import jax
import jax.numpy as jnp
from jax import lax
from jax.experimental import pallas as pl
from jax.experimental.pallas import tpu as pltpu

N_DEV = 4
BLK = 64
KMODE = os.environ.get("KMODE", "full")


def kernel(x, Wq, K_ext, V_ext, Wo):
    B, Sq, Dm = x.shape
    Skv_loc = K_ext.shape[1]
    Dh = K_ext.shape[3]
    Hl = Wq.shape[1] // Dh
    HD = Hl * Dh
    scale = 0.125

    Kb = K_ext.reshape(B, Skv_loc, K_ext.shape[2] * Dh).astype(jnp.bfloat16)
    Vb = V_ext.reshape(B, Skv_loc, V_ext.shape[2] * Dh).astype(jnp.bfloat16)

    def body(x_ref, wq_ref, k_ref, v_ref, wo_ref, out_ref,
             kbuf, vbuf, obuf, osend_stage, lsum, acc,
             copy_sems, ksend, krecv, vsend, vrecv, osend, orecv):
        my = lax.axis_index("i")

        kcp = pltpu.make_async_copy(
            k_ref.at[:, :, pl.ds(my * HD, HD)], kbuf.at[N_DEV - 1],
            copy_sems.at[0])
        vcp = pltpu.make_async_copy(
            v_ref.at[:, :, pl.ds(my * HD, HD)], vbuf.at[N_DEV - 1],
            copy_sems.at[1])
        kcp.start()
        vcp.start()

        kv_rdmas = []
        for off in range(1, N_DEV):
            d = lax.rem(my + off, N_DEV)
            kr = pltpu.make_async_remote_copy(
                src_ref=k_ref.at[:, :, pl.ds(d * HD, HD)],
                dst_ref=kbuf.at[off - 1],
                send_sem=ksend.at[off - 1], recv_sem=krecv.at[off - 1],
                device_id=(d,), device_id_type=pl.DeviceIdType.MESH)
            vr = pltpu.make_async_remote_copy(
                src_ref=v_ref.at[:, :, pl.ds(d * HD, HD)],
                dst_ref=vbuf.at[off - 1],
                send_sem=vsend.at[off - 1], recv_sem=vrecv.at[off - 1],
                device_id=(d,), device_id_type=pl.DeviceIdType.MESH)
            kr.start()
            vr.start()
            kv_rdmas.append((kr, vr))

        Qs = [jnp.dot(x_ref[b], wq_ref[...], preferred_element_type=jnp.float32)
              .astype(jnp.bfloat16) for b in range(B)]

        def fold_slot(slot, first):
            src = lax.rem(my + N_DEV - 1 - slot, N_DEV)
            qb = lax.broadcasted_iota(jnp.int32, (Sq, Skv_loc), 0) // BLK
            kj = lax.broadcasted_iota(jnp.int32, (Sq, Skv_loc), 1) + src * Skv_loc
            kb = kj // BLK
            mask = (qb == kb) | (kb == 0) | (lax.rem(qb + kb, 3) == 0)
            for b in range(B):
                for h in range(Hl):
                    q_h = Qs[b][:, h * Dh:(h + 1) * Dh]
                    sp = lax.dot_general(
                        q_h, kbuf[slot, b, :, h * Dh:(h + 1) * Dh],
                        (((1,), (1,)), ((), ())),
                        preferred_element_type=jnp.float32) * scale
                    e = jnp.where(mask, jnp.exp(sp), 0.0)
                    pv = jnp.dot(e.astype(jnp.bfloat16),
                                 vbuf[slot, b, :, h * Dh:(h + 1) * Dh],
                                 preferred_element_type=jnp.float32)
                    if first:
                        lsum[b, h] = jnp.sum(e, axis=1)
                        acc[b, h] = pv
                    else:
                        lsum[b, h] += jnp.sum(e, axis=1)
                        acc[b, h] += pv

        kcp.wait()
        vcp.wait()
        fold_slot(N_DEV - 1, first=True)
        for slot in (0, 2, 1):
            kv_rdmas[slot][0].wait_recv()
            kv_rdmas[slot][1].wait_recv()
            if KMODE != "nocompute":
                fold_slot(slot, first=False)

        send_out = KMODE != "noout"
        o_rdmas = []
        for b in range(B):
            ctx = jnp.concatenate(
                [acc[b, h] / lsum[b, h][:, None] for h in range(Hl)],
                axis=1)
            part = jnp.dot(ctx, wo_ref[...],
                           preferred_element_type=jnp.float32)
            out_ref[b] = part
            osend_stage[b] = part.astype(jnp.bfloat16)
            for off in range(1, N_DEV) if send_out else []:
                d = lax.rem(my + off, N_DEV)
                orr = pltpu.make_async_remote_copy(
                    src_ref=osend_stage.at[b],
                    dst_ref=obuf.at[off - 1, b],
                    send_sem=osend.at[b, off - 1],
                    recv_sem=orecv.at[b, off - 1],
                    device_id=(d,), device_id_type=pl.DeviceIdType.MESH)
                orr.start()
                o_rdmas.append(orr)

        for kr, vr in kv_rdmas:
            kr.wait_send()
            vr.wait_send()

        for orr in o_rdmas:
            orr.wait()

        if send_out:
            out_ref[...] = out_ref[...] + (
                obuf[0].astype(jnp.float32)
                + obuf[1].astype(jnp.float32)
                + obuf[2].astype(jnp.float32))

    return pl.pallas_call(
        body,
        out_shape=jax.ShapeDtypeStruct((B, Sq, Dm), jnp.float32),
        in_specs=[
            pl.BlockSpec(memory_space=pltpu.VMEM),
            pl.BlockSpec(memory_space=pltpu.VMEM),
            pl.BlockSpec(memory_space=pltpu.VMEM),
            pl.BlockSpec(memory_space=pltpu.VMEM),
            pl.BlockSpec(memory_space=pltpu.VMEM),
        ],
        out_specs=pl.BlockSpec(memory_space=pltpu.VMEM),
        scratch_shapes=[
            pltpu.VMEM((N_DEV, B, Skv_loc, HD), jnp.bfloat16),
            pltpu.VMEM((N_DEV, B, Skv_loc, HD), jnp.bfloat16),
            pltpu.VMEM((N_DEV - 1, B, Sq, Dm), jnp.bfloat16),
            pltpu.VMEM((B, Sq, Dm), jnp.bfloat16),
            pltpu.VMEM((B, Hl, Sq), jnp.float32),
            pltpu.VMEM((B, Hl, Sq, Dh), jnp.float32),
            pltpu.SemaphoreType.DMA((2,)),
            pltpu.SemaphoreType.DMA((N_DEV - 1,)),
            pltpu.SemaphoreType.DMA((N_DEV - 1,)),
            pltpu.SemaphoreType.DMA((N_DEV - 1,)),
            pltpu.SemaphoreType.DMA((N_DEV - 1,)),
            pltpu.SemaphoreType.DMA((B, N_DEV - 1)),
            pltpu.SemaphoreType.DMA((B, N_DEV - 1)),
        ],
        compiler_params=pltpu.CompilerParams(
            vmem_limit_bytes=64 * 1024 * 1024,
        ),
    )(x, Wq, Kb, Vb, Wo)


# device time: 126053 ns/iter; 2.7892x vs baseline; 1.0922x over previous
import os

import jax
import jax.numpy as jnp
from jax import lax
from jax.experimental import pallas as pl
from jax.experimental.pallas import tpu as pltpu

N_DEV = 4
BLK = 64
KMODE = os.environ.get("KMODE", "full")


def kernel(x, Wq, K_ext, V_ext, Wo):
    B, Sq, Dm = x.shape
    Skv_loc = K_ext.shape[1]
    Dh = K_ext.shape[3]
    Hl = Wq.shape[1] // Dh
    HD = Hl * Dh
    scale = 0.125

    Kb = K_ext.reshape(B, Skv_loc, K_ext.shape[2] * Dh).astype(jnp.bfloat16)
    Vb = V_ext.reshape(B, Skv_loc, V_ext.shape[2] * Dh).astype(jnp.bfloat16)

    def body(x_ref, wq_ref, k_ref, v_ref, wo_ref, out_ref,
             kbuf, vbuf, osend_stage, qrecv, qsend, gbuf, lsum, acc,
             copy_sems, ksend, krecv, vsend, vrecv,
             rs_send, rs_recv, ag_send, ag_recv):
        my = lax.axis_index("i")

        kcp = pltpu.make_async_copy(
            k_ref.at[:, :, pl.ds(my * HD, HD)], kbuf.at[N_DEV - 1],
            copy_sems.at[0])
        vcp = pltpu.make_async_copy(
            v_ref.at[:, :, pl.ds(my * HD, HD)], vbuf.at[N_DEV - 1],
            copy_sems.at[1])
        kcp.start()
        vcp.start()

        kv_rdmas = []
        for off in range(1, N_DEV):
            d = lax.rem(my + off, N_DEV)
            kr = pltpu.make_async_remote_copy(
                src_ref=k_ref.at[:, :, pl.ds(d * HD, HD)],
                dst_ref=kbuf.at[off - 1],
                send_sem=ksend.at[off - 1], recv_sem=krecv.at[off - 1],
                device_id=(d,), device_id_type=pl.DeviceIdType.MESH)
            vr = pltpu.make_async_remote_copy(
                src_ref=v_ref.at[:, :, pl.ds(d * HD, HD)],
                dst_ref=vbuf.at[off - 1],
                send_sem=vsend.at[off - 1], recv_sem=vrecv.at[off - 1],
                device_id=(d,), device_id_type=pl.DeviceIdType.MESH)
            kr.start()
            vr.start()
            kv_rdmas.append((kr, vr))

        Qs = [jnp.dot(x_ref[b], wq_ref[...], preferred_element_type=jnp.float32)
              .astype(jnp.bfloat16) for b in range(B)]

        def fold_slot(slot, first):
            src = lax.rem(my + N_DEV - 1 - slot, N_DEV)
            qb = lax.broadcasted_iota(jnp.int32, (Sq, Skv_loc), 0) // BLK
            kj = lax.broadcasted_iota(jnp.int32, (Sq, Skv_loc), 1) + src * Skv_loc
            kb = kj // BLK
            mask = (qb == kb) | (kb == 0) | (lax.rem(qb + kb, 3) == 0)
            for b in range(B):
                for h in range(Hl):
                    q_h = Qs[b][:, h * Dh:(h + 1) * Dh]
                    sp = lax.dot_general(
                        q_h, kbuf[slot, b, :, h * Dh:(h + 1) * Dh],
                        (((1,), (1,)), ((), ())),
                        preferred_element_type=jnp.float32) * scale
                    e = jnp.where(mask, jnp.exp(sp), 0.0)
                    pv = jnp.dot(e.astype(jnp.bfloat16),
                                 vbuf[slot, b, :, h * Dh:(h + 1) * Dh],
                                 preferred_element_type=jnp.float32)
                    if first:
                        lsum[b, h] = jnp.sum(e, axis=1)
                        acc[b, h] = pv
                    else:
                        lsum[b, h] += jnp.sum(e, axis=1)
                        acc[b, h] += pv

        kcp.wait()
        vcp.wait()
        fold_slot(N_DEV - 1, first=True)
        for slot in (0, 2, 1):
            kv_rdmas[slot][0].wait_recv()
            kv_rdmas[slot][1].wait_recv()
            if KMODE != "nocompute":
                fold_slot(slot, first=False)

        for b in range(B):
            ctx = jnp.concatenate(
                [acc[b, h] / lsum[b, h][:, None] for h in range(Hl)],
                axis=1)
            part = jnp.dot(ctx, wo_ref[...],
                           preferred_element_type=jnp.float32)
            out_ref[b] = part
            osend_stage[b] = part.astype(jnp.bfloat16)

        SqH = Sq // 2
        send_out = KMODE != "noout"
        my_qb = my // 2
        my_qr = lax.rem(my, 2)

        rs = []
        for off in range(1, N_DEV) if send_out else []:
            d = lax.rem(my + off, N_DEV)
            r = pltpu.make_async_remote_copy(
                src_ref=osend_stage.at[d // 2, pl.ds(lax.rem(d, 2) * SqH, SqH)],
                dst_ref=qrecv.at[off - 1],
                send_sem=rs_send.at[off - 1], recv_sem=rs_recv.at[off - 1],
                device_id=(d,), device_id_type=pl.DeviceIdType.MESH)
            r.start()
            rs.append(r)

        for kr, vr in kv_rdmas:
            kr.wait_send()
            vr.wait_send()

        if send_out:
            for r in rs:
                r.wait()
            myq = out_ref[my_qb, pl.ds(my_qr * SqH, SqH), :]
            total = (myq
                     + qrecv[0].astype(jnp.float32)
                     + qrecv[1].astype(jnp.float32)
                     + qrecv[2].astype(jnp.float32))
            out_ref[my_qb, pl.ds(my_qr * SqH, SqH), :] = total
            qsend[...] = total.astype(jnp.bfloat16)

            ag = []
            for off in range(1, N_DEV):
                d = lax.rem(my + off, N_DEV)
                a = pltpu.make_async_remote_copy(
                    src_ref=qsend,
                    dst_ref=gbuf.at[off - 1],
                    send_sem=ag_send.at[off - 1], recv_sem=ag_recv.at[off - 1],
                    device_id=(d,), device_id_type=pl.DeviceIdType.MESH)
                a.start()
                ag.append(a)
            for a in ag:
                a.wait()
            for off in range(1, N_DEV):
                s = lax.rem(my + N_DEV - off, N_DEV)
                out_ref[s // 2, pl.ds(lax.rem(s, 2) * SqH, SqH), :] = (
                    gbuf[off - 1].astype(jnp.float32))

    return pl.pallas_call(
        body,
        out_shape=jax.ShapeDtypeStruct((B, Sq, Dm), jnp.float32),
        in_specs=[
            pl.BlockSpec(memory_space=pltpu.VMEM),
            pl.BlockSpec(memory_space=pltpu.VMEM),
            pl.BlockSpec(memory_space=pltpu.VMEM),
            pl.BlockSpec(memory_space=pltpu.VMEM),
            pl.BlockSpec(memory_space=pltpu.VMEM),
        ],
        out_specs=pl.BlockSpec(memory_space=pltpu.VMEM),
        scratch_shapes=[
            pltpu.VMEM((N_DEV, B, Skv_loc, HD), jnp.bfloat16),
            pltpu.VMEM((N_DEV, B, Skv_loc, HD), jnp.bfloat16),
            pltpu.VMEM((B, Sq, Dm), jnp.bfloat16),
            pltpu.VMEM((N_DEV - 1, Sq // 2, Dm), jnp.bfloat16),
            pltpu.VMEM((Sq // 2, Dm), jnp.bfloat16),
            pltpu.VMEM((N_DEV - 1, Sq // 2, Dm), jnp.bfloat16),
            pltpu.VMEM((B, Hl, Sq), jnp.float32),
            pltpu.VMEM((B, Hl, Sq, Dh), jnp.float32),
            pltpu.SemaphoreType.DMA((2,)),
            pltpu.SemaphoreType.DMA((N_DEV - 1,)),
            pltpu.SemaphoreType.DMA((N_DEV - 1,)),
            pltpu.SemaphoreType.DMA((N_DEV - 1,)),
            pltpu.SemaphoreType.DMA((N_DEV - 1,)),
            pltpu.SemaphoreType.DMA((N_DEV - 1,)),
            pltpu.SemaphoreType.DMA((N_DEV - 1,)),
            pltpu.SemaphoreType.DMA((N_DEV - 1,)),
            pltpu.SemaphoreType.DMA((N_DEV - 1,)),
        ],
        compiler_params=pltpu.CompilerParams(
            vmem_limit_bytes=64 * 1024 * 1024,
        ),
    )(x, Wq, Kb, Vb, Wo)


# device time: 122499 ns/iter; 2.8701x vs baseline; 1.0290x over previous
import os

import jax
import jax.numpy as jnp
from jax import lax
from jax.experimental import pallas as pl
from jax.experimental.pallas import tpu as pltpu

N_DEV = 4
BLK = 64
KMODE = os.environ.get("KMODE", "full")


def kernel(x, Wq, K_ext, V_ext, Wo):
    B, Sq, Dm = x.shape
    Skv_loc = K_ext.shape[1]
    Dh = K_ext.shape[3]
    Hl = Wq.shape[1] // Dh
    HD = Hl * Dh
    scale = 0.125

    Kb = K_ext.reshape(B, Skv_loc, K_ext.shape[2] * Dh).astype(jnp.bfloat16)
    Vb = V_ext.reshape(B, Skv_loc, V_ext.shape[2] * Dh).astype(jnp.bfloat16)

    def body(x_ref, wq_ref, k_ref, v_ref, wo_ref, out_ref,
             kbuf, vbuf, osend_stage, qrecv, qsend, gbuf, lsum, acc,
             copy_sems, ksend, krecv, vsend, vrecv,
             rs_send, rs_recv, ag_send, ag_recv):
        my = lax.axis_index("i")

        kcp = pltpu.make_async_copy(
            k_ref.at[:, :, pl.ds(my * HD, HD)], kbuf.at[N_DEV - 1],
            copy_sems.at[0])
        vcp = pltpu.make_async_copy(
            v_ref.at[:, :, pl.ds(my * HD, HD)], vbuf.at[N_DEV - 1],
            copy_sems.at[1])
        kcp.start()
        vcp.start()

        kv_rdmas = []
        for off in range(1, N_DEV):
            d = lax.rem(my + off, N_DEV)
            kr = pltpu.make_async_remote_copy(
                src_ref=k_ref.at[:, :, pl.ds(d * HD, HD)],
                dst_ref=kbuf.at[off - 1],
                send_sem=ksend.at[off - 1], recv_sem=krecv.at[off - 1],
                device_id=(d,), device_id_type=pl.DeviceIdType.MESH)
            vr = pltpu.make_async_remote_copy(
                src_ref=v_ref.at[:, :, pl.ds(d * HD, HD)],
                dst_ref=vbuf.at[off - 1],
                send_sem=vsend.at[off - 1], recv_sem=vrecv.at[off - 1],
                device_id=(d,), device_id_type=pl.DeviceIdType.MESH)
            kr.start()
            vr.start()
            kv_rdmas.append((kr, vr))

        Qs = [jnp.dot(x_ref[b], wq_ref[...], preferred_element_type=jnp.float32)
              .astype(jnp.bfloat16) for b in range(B)]

        def fold_slot(slot, first):
            src = lax.rem(my + N_DEV - 1 - slot, N_DEV)
            qb = lax.broadcasted_iota(jnp.int32, (Sq, Skv_loc), 0) // BLK
            kj = lax.broadcasted_iota(jnp.int32, (Sq, Skv_loc), 1) + src * Skv_loc
            kb = kj // BLK
            mask = (qb == kb) | (kb == 0) | (lax.rem(qb + kb, 3) == 0)
            for b in range(B):
                for h in range(Hl):
                    q_h = Qs[b][:, h * Dh:(h + 1) * Dh]
                    sp = lax.dot_general(
                        q_h, kbuf[slot, b, :, h * Dh:(h + 1) * Dh],
                        (((1,), (1,)), ((), ())),
                        preferred_element_type=jnp.float32) * scale
                    e = jnp.where(mask, jnp.exp(sp), 0.0)
                    pv = jnp.dot(e.astype(jnp.bfloat16),
                                 vbuf[slot, b, :, h * Dh:(h + 1) * Dh],
                                 preferred_element_type=jnp.float32)
                    if first:
                        lsum[b, h] = jnp.sum(e, axis=1)
                        acc[b, h] = pv
                    else:
                        lsum[b, h] += jnp.sum(e, axis=1)
                        acc[b, h] += pv

        kcp.wait()
        vcp.wait()
        fold_slot(N_DEV - 1, first=True)
        for slot in (0, 2, 1):
            kv_rdmas[slot][0].wait_recv()
            kv_rdmas[slot][1].wait_recv()
            if KMODE != "nocompute":
                fold_slot(slot, first=False)

        SqH = Sq // 2
        send_out = KMODE != "noout"
        my_qb = my // 2
        my_qr = lax.rem(my, 2)

        rs = []
        for off in range(1, N_DEV) if send_out else []:
            d = lax.rem(my + off, N_DEV)
            rs.append((pltpu.make_async_remote_copy(
                src_ref=osend_stage.at[d // 2, pl.ds(lax.rem(d, 2) * SqH, SqH)],
                dst_ref=qrecv.at[off - 1],
                send_sem=rs_send.at[off - 1], recv_sem=rs_recv.at[off - 1],
                device_id=(d,), device_id_type=pl.DeviceIdType.MESH), d))

        wo_b16 = wo_ref[...].astype(jnp.bfloat16)
        for b in range(B):
            ctx = jnp.concatenate(
                [(acc[b, h] / lsum[b, h][:, None]).astype(jnp.bfloat16)
                 for h in range(Hl)],
                axis=1)
            part = jnp.dot(ctx, wo_b16,
                           preferred_element_type=jnp.float32)
            out_ref[b] = part
            osend_stage[b] = part.astype(jnp.bfloat16)
            for r, d in rs:
                @pl.when(d // 2 == b)
                def _():
                    r.start()

        for kr, vr in kv_rdmas:
            kr.wait_send()
            vr.wait_send()

        if send_out:
            for r, _d in rs:
                r.wait()
            myq = out_ref[my_qb, pl.ds(my_qr * SqH, SqH), :]
            total = (myq
                     + qrecv[0].astype(jnp.float32)
                     + qrecv[1].astype(jnp.float32)
                     + qrecv[2].astype(jnp.float32))
            out_ref[my_qb, pl.ds(my_qr * SqH, SqH), :] = total
            qsend[...] = total.astype(jnp.bfloat16)

            ag = []
            for off in range(1, N_DEV):
                d = lax.rem(my + off, N_DEV)
                a = pltpu.make_async_remote_copy(
                    src_ref=qsend,
                    dst_ref=gbuf.at[off - 1],
                    send_sem=ag_send.at[off - 1], recv_sem=ag_recv.at[off - 1],
                    device_id=(d,), device_id_type=pl.DeviceIdType.MESH)
                a.start()
                ag.append(a)
            for a in ag:
                a.wait()
            for off in range(1, N_DEV):
                s = lax.rem(my + N_DEV - off, N_DEV)
                out_ref[s // 2, pl.ds(lax.rem(s, 2) * SqH, SqH), :] = (
                    gbuf[off - 1].astype(jnp.float32))

    return pl.pallas_call(
        body,
        out_shape=jax.ShapeDtypeStruct((B, Sq, Dm), jnp.float32),
        in_specs=[
            pl.BlockSpec(memory_space=pltpu.VMEM),
            pl.BlockSpec(memory_space=pltpu.VMEM),
            pl.BlockSpec(memory_space=pltpu.VMEM),
            pl.BlockSpec(memory_space=pltpu.VMEM),
            pl.BlockSpec(memory_space=pltpu.VMEM),
        ],
        out_specs=pl.BlockSpec(memory_space=pltpu.VMEM),
        scratch_shapes=[
            pltpu.VMEM((N_DEV, B, Skv_loc, HD), jnp.bfloat16),
            pltpu.VMEM((N_DEV, B, Skv_loc, HD), jnp.bfloat16),
            pltpu.VMEM((B, Sq, Dm), jnp.bfloat16),
            pltpu.VMEM((N_DEV - 1, Sq // 2, Dm), jnp.bfloat16),
            pltpu.VMEM((Sq // 2, Dm), jnp.bfloat16),
            pltpu.VMEM((N_DEV - 1, Sq // 2, Dm), jnp.bfloat16),
            pltpu.VMEM((B, Hl, Sq), jnp.float32),
            pltpu.VMEM((B, Hl, Sq, Dh), jnp.float32),
            pltpu.SemaphoreType.DMA((2,)),
            pltpu.SemaphoreType.DMA((N_DEV - 1,)),
            pltpu.SemaphoreType.DMA((N_DEV - 1,)),
            pltpu.SemaphoreType.DMA((N_DEV - 1,)),
            pltpu.SemaphoreType.DMA((N_DEV - 1,)),
            pltpu.SemaphoreType.DMA((N_DEV - 1,)),
            pltpu.SemaphoreType.DMA((N_DEV - 1,)),
            pltpu.SemaphoreType.DMA((N_DEV - 1,)),
            pltpu.SemaphoreType.DMA((N_DEV - 1,)),
        ],
        compiler_params=pltpu.CompilerParams(
            vmem_limit_bytes=64 * 1024 * 1024,
        ),
    )(x, Wq, Kb, Vb, Wo)
